# Initial kernel scaffold; baseline (speedup 1.0000x reference)
import jax
import jax.numpy as jnp
from jax.experimental import pallas as pl


def kernel(num, pe):
    raise NotImplementedError("write your pallas kernel here")


# speedup vs baseline: 7.0217x; 7.0217x over previous
import functools

import jax
import jax.numpy as jnp
from jax import lax
from jax.experimental import pallas as pl
from jax.experimental.pallas import tpu as pltpu
from jax.experimental.pallas import tpu_sc as plsc

DIM = 128
NC = 2
NS = 16
NW = NC * NS
CHUNK = 128


def _sc_gather(num3, pe, nchunk):
    total = NW * nchunk * CHUNK
    mesh = plsc.VectorSubcoreMesh(core_axis_name="c", subcore_axis_name="s")

    @functools.partial(
        pl.kernel,
        mesh=mesh,
        out_type=jax.ShapeDtypeStruct((total, DIM), jnp.float32),
        scratch_types=[
            pltpu.VMEM((nchunk, CHUNK), jnp.int32),
            pltpu.VMEM((CHUNK, DIM), jnp.float32),
            pltpu.SemaphoreType.DMA,
        ],
    )
    def k(idx_hbm, pe_hbm, out_hbm, idx_v, rows_v, sem):
        wid = lax.axis_index("s") * NC + lax.axis_index("c")
        base = wid * (nchunk * CHUNK)
        pltpu.sync_copy(idx_hbm.at[wid], idx_v)

        def body(j, carry):
            pltpu.async_copy(pe_hbm.at[idx_v.at[j]], rows_v, sem).wait()
            pltpu.sync_copy(rows_v, out_hbm.at[pl.ds(base + j * CHUNK, CHUNK)])
            return carry

        lax.fori_loop(0, nchunk, body, 0)

    return k(num3, pe)


def kernel(num, pe):
    batch, hist = num.shape
    total = batch * hist
    nchunk = total // (NW * CHUNK)
    num3 = num.reshape(NW, nchunk, CHUNK).astype(jnp.int32)
    out = _sc_gather(num3, pe, nchunk)
    return out.reshape(batch, hist, DIM)


# speedup vs baseline: 9.9652x; 1.4192x over previous
import functools

import jax
import jax.numpy as jnp
from jax import lax
from jax.experimental import pallas as pl
from jax.experimental.pallas import tpu as pltpu
from jax.experimental.pallas import tpu_sc as plsc

DIM = 128
NC = 2
NS = 16
NW = NC * NS
CHUNK = 128
NBUF = 4


def _sc_gather(num3, pe, nchunk):
    total = NW * nchunk * CHUNK
    ngroups = nchunk // NBUF
    mesh = plsc.VectorSubcoreMesh(core_axis_name="c", subcore_axis_name="s")

    scratch = (
        [pltpu.VMEM((nchunk, CHUNK), jnp.int32)]
        + [pltpu.VMEM((CHUNK, DIM), jnp.float32) for _ in range(NBUF)]
        + [pltpu.SemaphoreType.DMA for _ in range(2 * NBUF)]
    )

    @functools.partial(
        pl.kernel,
        mesh=mesh,
        out_type=jax.ShapeDtypeStruct((total, DIM), jnp.float32),
        scratch_types=scratch,
    )
    def k(idx_hbm, pe_hbm, out_hbm, *refs):
        idx_v = refs[0]
        rows = refs[1:1 + NBUF]
        sem_g = refs[1 + NBUF:1 + 2 * NBUF]
        sem_o = refs[1 + 2 * NBUF:1 + 3 * NBUF]

        wid = lax.axis_index("s") * NC + lax.axis_index("c")
        base = wid * (nchunk * CHUNK)
        pltpu.sync_copy(idx_hbm.at[wid], idx_v)

        for b in range(NBUF):
            pltpu.async_copy(pe_hbm.at[idx_v.at[b]], rows[b], sem_g[b])

        def group(g, carry):
            for b in range(NBUF):
                j = g * NBUF + b
                pltpu.make_async_copy(
                    pe_hbm.at[pl.ds(0, CHUNK)], rows[b], sem_g[b]
                ).wait()
                pltpu.async_copy(
                    rows[b], out_hbm.at[pl.ds(base + j * CHUNK, CHUNK)],
                    sem_o[b],
                )
            @pl.when(g + 1 < ngroups)
            def _():
                for b in range(NBUF):
                    jn = (g + 1) * NBUF + b
                    pltpu.make_async_copy(
                        rows[b], out_hbm.at[pl.ds(base, CHUNK)], sem_o[b]
                    ).wait()
                    pltpu.async_copy(
                        pe_hbm.at[idx_v.at[jn]], rows[b], sem_g[b]
                    )
            return carry

        lax.fori_loop(0, ngroups, group, 0)

        for b in range(NBUF):
            pltpu.make_async_copy(
                rows[b], out_hbm.at[pl.ds(base, CHUNK)], sem_o[b]
            ).wait()

    return k(num3, pe)


def kernel(num, pe):
    batch, hist = num.shape
    total = batch * hist
    nchunk = total // (NW * CHUNK)
    num3 = num.reshape(NW, nchunk, CHUNK).astype(jnp.int32)
    out = _sc_gather(num3, pe, nchunk)
    return out.reshape(batch, hist, DIM)


# speedup vs baseline: 10.0701x; 1.0105x over previous
import functools

import jax
import jax.numpy as jnp
from jax import lax
from jax.experimental import pallas as pl
from jax.experimental.pallas import tpu as pltpu
from jax.experimental.pallas import tpu_sc as plsc

DIM = 128
NC = 2
NS = 16
NW = NC * NS
CHUNK = 128
NBUF = 5


def _sc_gather(num3, pe, nchunk):
    total = NW * nchunk * CHUNK
    ngroups = nchunk // NBUF
    mesh = plsc.VectorSubcoreMesh(core_axis_name="c", subcore_axis_name="s")

    scratch = (
        [pltpu.VMEM((nchunk, CHUNK), jnp.int32)]
        + [pltpu.VMEM((CHUNK, DIM), jnp.float32) for _ in range(NBUF)]
        + [pltpu.SemaphoreType.DMA for _ in range(2 * NBUF)]
    )

    @functools.partial(
        pl.kernel,
        mesh=mesh,
        out_type=jax.ShapeDtypeStruct((total, DIM), jnp.float32),
        scratch_types=scratch,
    )
    def k(idx_hbm, pe_hbm, out_hbm, *refs):
        idx_v = refs[0]
        rows = refs[1:1 + NBUF]
        sem_g = refs[1 + NBUF:1 + 2 * NBUF]
        sem_o = refs[1 + 2 * NBUF:1 + 3 * NBUF]

        wid = lax.axis_index("s") * NC + lax.axis_index("c")
        base = wid * (nchunk * CHUNK)
        pltpu.sync_copy(idx_hbm.at[wid], idx_v)

        for b in range(NBUF):
            pltpu.async_copy(pe_hbm.at[idx_v.at[b]], rows[b], sem_g[b])

        def group(g, carry):
            for b in range(NBUF):
                j = g * NBUF + b
                pltpu.make_async_copy(
                    pe_hbm.at[pl.ds(0, CHUNK)], rows[b], sem_g[b]
                ).wait()
                pltpu.async_copy(
                    rows[b], out_hbm.at[pl.ds(base + j * CHUNK, CHUNK)],
                    sem_o[b],
                )
            @pl.when(g + 1 < ngroups)
            def _():
                for b in range(NBUF):
                    jn = (g + 1) * NBUF + b
                    pltpu.make_async_copy(
                        rows[b], out_hbm.at[pl.ds(base, CHUNK)], sem_o[b]
                    ).wait()
                    pltpu.async_copy(
                        pe_hbm.at[idx_v.at[jn]], rows[b], sem_g[b]
                    )
            return carry

        lax.fori_loop(0, ngroups, group, 0)

        for b in range(NBUF):
            pltpu.make_async_copy(
                rows[b], out_hbm.at[pl.ds(base, CHUNK)], sem_o[b]
            ).wait()

        for j in range(ngroups * NBUF, nchunk):
            pltpu.async_copy(pe_hbm.at[idx_v.at[j]], rows[0], sem_g[0]).wait()
            pltpu.sync_copy(rows[0], out_hbm.at[pl.ds(base + j * CHUNK, CHUNK)])

    return k(num3, pe)


def kernel(num, pe):
    batch, hist = num.shape
    total = batch * hist
    nchunk = total // (NW * CHUNK)
    num3 = num.reshape(NW, nchunk, CHUNK).astype(jnp.int32)
    out = _sc_gather(num3, pe, nchunk)
    return out.reshape(batch, hist, DIM)
